# Initial kernel scaffold; baseline (speedup 1.0000x reference)
import jax
import jax.numpy as jnp
from jax.experimental import pallas as pl


def kernel(x, edge_index, W1, b1, W_mu, b_mu, W_var, b_var):
    raise NotImplementedError("write your pallas kernel here")


# speedup vs baseline: 151.3425x; 151.3425x over previous
import functools

import jax
import jax.numpy as jnp
from jax import lax
from jax.experimental import pallas as pl
from jax.experimental.pallas import tpu as pltpu
from jax.experimental.pallas import tpu_sc as plsc

N = 10000
D = 128
NC = 2
NS = 16
NW = NC * NS
NB = 80
BATCH = 128
EPW = NB * BATCH
EP = NW * EPW
NP = 10240
RPT = NP // NS
BLK = 256
GRID = NP // BLK


def _sc_mesh():
    return plsc.VectorSubcoreMesh(
        core_axis_name="c", subcore_axis_name="s",
        num_cores=NC, num_subcores=NS)


def _deg_body(dst_hbm, out0, out1, dst_v, zbuf, ones_v, acc):
    c = lax.axis_index("c")
    s = lax.axis_index("s")
    wid = c * jnp.int32(NS) + s

    def fill_z(i, carry):
        zbuf[pl.ds(i * jnp.int32(16), 16)] = jnp.zeros((16,), jnp.float32)
        return carry

    lax.fori_loop(jnp.int32(0), jnp.int32(RPT // 16), fill_z, 0)

    def fill_o(i, carry):
        ones_v[pl.ds(i * jnp.int32(16), 16)] = jnp.ones((16,), jnp.float32)
        return carry

    lax.fori_loop(jnp.int32(0), jnp.int32(BATCH // 16), fill_o, 0)

    rows = pl.ds(s * jnp.int32(RPT), RPT)
    pltpu.sync_copy(dst_hbm.at[wid], dst_v)
    pltpu.sync_copy(zbuf, acc.at[rows])
    plsc.subcore_barrier()

    def body(j, carry):
        pltpu.sync_copy(ones_v, acc.at[dst_v.at[j]], add=True)
        return carry

    lax.fori_loop(jnp.int32(0), jnp.int32(NB), body, 0)
    plsc.subcore_barrier()

    @pl.when(c == 0)
    def _():
        pltpu.sync_copy(acc.at[rows], out0.at[rows])

    @pl.when(c == 1)
    def _():
        pltpu.sync_copy(acc.at[rows], out1.at[rows])


_deg_call = functools.partial(
    pl.kernel,
    out_type=(
        jax.ShapeDtypeStruct((NP,), jnp.float32),
        jax.ShapeDtypeStruct((NP,), jnp.float32),
    ),
    mesh=_sc_mesh(),
    scratch_types=[
        pltpu.VMEM((NB, BATCH), jnp.int32),
        pltpu.VMEM((RPT,), jnp.float32),
        pltpu.VMEM((BATCH,), jnp.float32),
        pltpu.VMEM_SHARED((NP,), jnp.float32),
    ],
)(_deg_body)


def _agg_body(h_hbm, src_hbm, dst_hbm, out0, out1, src_v, dst_v, buf, acc, sem):
    c = lax.axis_index("c")
    s = lax.axis_index("s")
    wid = c * jnp.int32(NS) + s

    pltpu.sync_copy(src_hbm.at[wid], src_v)
    pltpu.sync_copy(dst_hbm.at[wid], dst_v)
    rows = pl.ds(s * jnp.int32(RPT), RPT)
    pltpu.sync_copy(h_hbm.at[rows], acc.at[rows])
    plsc.subcore_barrier()

    def body(j, carry):
        pltpu.async_copy(h_hbm.at[src_v.at[j]], buf, sem).wait()
        pltpu.sync_copy(buf, acc.at[dst_v.at[j]], add=True)
        return carry

    lax.fori_loop(jnp.int32(0), jnp.int32(NB), body, 0)
    plsc.subcore_barrier()

    @pl.when(c == 0)
    def _():
        pltpu.sync_copy(acc.at[rows], out0.at[rows])

    @pl.when(c == 1)
    def _():
        pltpu.sync_copy(acc.at[rows], out1.at[rows])


_agg_call = functools.partial(
    pl.kernel,
    out_type=(
        jax.ShapeDtypeStruct((NP, D), jnp.float32),
        jax.ShapeDtypeStruct((NP, D), jnp.float32),
    ),
    mesh=_sc_mesh(),
    scratch_types=[
        pltpu.VMEM((NB, BATCH), jnp.int32),
        pltpu.VMEM((NB, BATCH), jnp.int32),
        pltpu.VMEM((BATCH, D), jnp.float32),
        pltpu.VMEM_SHARED((NP, D), jnp.float32),
        pltpu.SemaphoreType.DMA,
    ],
)(_agg_body)


def _tc1_body(deg0_ref, deg1_ref, x_ref, w_ref, h_ref, dinv_ref):
    d = deg0_ref[0, 0, :] + deg1_ref[0, 0, :] + 1.0
    di = lax.rsqrt(d)
    h = jnp.dot(x_ref[...], w_ref[...], preferred_element_type=jnp.float32,
                precision=lax.Precision.HIGHEST)
    h_ref[...] = di[:, None] * h
    dinv_ref[0, 0, :] = di


def _tc2_body(s0_ref, s1_ref, hp_ref, dinv_ref, b_ref, out_ref):
    di = dinv_ref[0, 0, :][:, None]
    s = s0_ref[...] + s1_ref[...] - hp_ref[...]
    h = jnp.maximum(di * s + b_ref[...][None, :], 0.0)
    out_ref[...] = di * h


def _tc3_body(s0_ref, s1_ref, hp_ref, dinv_ref, w_ref, b_ref, out_ref):
    di = dinv_ref[0, 0, :][:, None]
    a = di * (s0_ref[...] + s1_ref[...] - hp_ref[...])
    out_ref[...] = (
        jnp.dot(a, w_ref[...], preferred_element_type=jnp.float32,
                precision=lax.Precision.HIGHEST)
        + b_ref[...][None, :]
    )


def _row_spec(width):
    return pl.BlockSpec((BLK, width), lambda i: (i, 0))


def _vec_spec():
    return pl.BlockSpec((1, 1, BLK), lambda i: (i, 0, 0))


def _full_spec(r, c):
    return pl.BlockSpec((r, c), lambda i: (0, 0))


_tc1_call = pl.pallas_call(
    _tc1_body,
    grid=(GRID,),
    in_specs=[_vec_spec(), _vec_spec(), _row_spec(D), _full_spec(D, D)],
    out_specs=[_row_spec(D), _vec_spec()],
    out_shape=[
        jax.ShapeDtypeStruct((NP, D), jnp.float32),
        jax.ShapeDtypeStruct((GRID, 1, BLK), jnp.float32),
    ],
)

_tc2_call = pl.pallas_call(
    _tc2_body,
    grid=(GRID,),
    in_specs=[
        _row_spec(D),
        _row_spec(D),
        _row_spec(D),
        _vec_spec(),
        pl.BlockSpec((D,), lambda i: (0,)),
    ],
    out_specs=_row_spec(D),
    out_shape=jax.ShapeDtypeStruct((NP, D), jnp.float32),
)

_tc3_call = pl.pallas_call(
    _tc3_body,
    grid=(GRID,),
    in_specs=[
        _row_spec(D),
        _row_spec(D),
        _row_spec(D),
        _vec_spec(),
        _full_spec(D, D),
        pl.BlockSpec((D,), lambda i: (0,)),
    ],
    out_specs=_row_spec(D),
    out_shape=jax.ShapeDtypeStruct((NP, D), jnp.float32),
)


@jax.jit
def _run(x, src, dst, W1, b1, W_cat, b_cat):
    pad = EP - src.shape[0]
    src_p = jnp.concatenate(
        [src, jnp.zeros((pad,), jnp.int32)]).reshape(NW, NB, BATCH)
    dst_p = jnp.concatenate(
        [dst, jnp.full((pad,), N, jnp.int32)]).reshape(NW, NB, BATCH)
    xp = jnp.zeros((NP, D), jnp.float32).at[:N].set(x)

    deg0, deg1 = _deg_call(dst_p)
    hp, dinv = _tc1_call(
        deg0.reshape(GRID, 1, BLK), deg1.reshape(GRID, 1, BLK), xp, W1)
    s1a, s1b = _agg_call(hp, src_p, dst_p)
    hp2 = _tc2_call(s1a, s1b, hp, dinv, b1)
    s2a, s2b = _agg_call(hp2, src_p, dst_p)
    out = _tc3_call(s2a, s2b, hp2, dinv, W_cat, b_cat)
    return out[:N, :64], out[:N, 64:]


def kernel(x, edge_index, W1, b1, W_mu, b_mu, W_var, b_var):
    with jax.enable_x64(False):
        src = edge_index[0].astype(jnp.int32)
        dst = edge_index[1].astype(jnp.int32)
        W_cat = jnp.concatenate([W_mu, W_var], axis=1)
        b_cat = jnp.concatenate([b_mu, b_var], axis=0)
        mu, lv = _run(x.astype(jnp.float32), src, dst,
                      W1.astype(jnp.float32), b1.astype(jnp.float32),
                      W_cat.astype(jnp.float32), b_cat.astype(jnp.float32))
    return mu.astype(jnp.float64), lv.astype(jnp.float64)


# speedup vs baseline: 166.3059x; 1.0989x over previous
import functools

import jax
import jax.numpy as jnp
from jax import lax
from jax.experimental import pallas as pl
from jax.experimental.pallas import tpu as pltpu
from jax.experimental.pallas import tpu_sc as plsc

N = 10000
D = 128
NC = 2
NS = 16
NW = NC * NS
NB = 80
BATCH = 128
EPW = NB * BATCH
EP = NW * EPW
NP = 10240
RPT = NP // NS
BLK = 256
GRID = NP // BLK


def _sc_mesh():
    return plsc.VectorSubcoreMesh(
        core_axis_name="c", subcore_axis_name="s",
        num_cores=NC, num_subcores=NS)


def _deg_body(dst_hbm, out0, out1, dst_v, zbuf, ones_v, acc):
    c = lax.axis_index("c")
    s = lax.axis_index("s")
    wid = c * jnp.int32(NS) + s

    def fill_z(i, carry):
        zbuf[pl.ds(i * jnp.int32(16), 16)] = jnp.zeros((16,), jnp.float32)
        return carry

    lax.fori_loop(jnp.int32(0), jnp.int32(RPT // 16), fill_z, 0)

    def fill_o(i, carry):
        ones_v[pl.ds(i * jnp.int32(16), 16)] = jnp.ones((16,), jnp.float32)
        return carry

    lax.fori_loop(jnp.int32(0), jnp.int32(BATCH // 16), fill_o, 0)

    rows = pl.ds(s * jnp.int32(RPT), RPT)
    pltpu.sync_copy(dst_hbm.at[wid], dst_v)
    pltpu.sync_copy(zbuf, acc.at[rows])
    plsc.subcore_barrier()

    def body(j, carry):
        pltpu.sync_copy(ones_v, acc.at[dst_v.at[j]], add=True)
        return carry

    lax.fori_loop(jnp.int32(0), jnp.int32(NB), body, 0)
    plsc.subcore_barrier()

    @pl.when(c == 0)
    def _():
        pltpu.sync_copy(acc.at[rows], out0.at[rows])

    @pl.when(c == 1)
    def _():
        pltpu.sync_copy(acc.at[rows], out1.at[rows])


_deg_call = functools.partial(
    pl.kernel,
    out_type=(
        jax.ShapeDtypeStruct((NP,), jnp.float32),
        jax.ShapeDtypeStruct((NP,), jnp.float32),
    ),
    mesh=_sc_mesh(),
    scratch_types=[
        pltpu.VMEM((NB, BATCH), jnp.int32),
        pltpu.VMEM((RPT,), jnp.float32),
        pltpu.VMEM((BATCH,), jnp.float32),
        pltpu.VMEM_SHARED((NP,), jnp.float32),
    ],
)(_deg_body)


CH = 16
NCHUNK = NB // CH


def _agg_body(h_hbm, src_hbm, dst_hbm, out0, out1,
              src_v, dst_v, buf, acc, gsem, isem):
    c = lax.axis_index("c")
    s = lax.axis_index("s")
    wid = c * jnp.int32(NS) + s
    i32 = jnp.int32

    pltpu.sync_copy(src_hbm.at[wid, pl.ds(0, CH)], src_v.at[i32(0)])
    pltpu.sync_copy(dst_hbm.at[wid, pl.ds(0, CH)], dst_v.at[i32(0)])
    rows = pl.ds(s * i32(RPT), RPT)
    pltpu.sync_copy(h_hbm.at[rows], acc.at[rows])
    pltpu.async_copy(h_hbm.at[src_v.at[i32(0), i32(0)]],
                     buf.at[i32(0)], gsem.at[i32(0)])
    plsc.subcore_barrier()

    for k in range(NCHUNK):
        pk = i32(k % 2)
        pn = i32((k + 1) % 2)
        if k + 1 < NCHUNK:
            ia = pltpu.async_copy(
                src_hbm.at[wid, pl.ds((k + 1) * CH, CH)], src_v.at[pn], isem)
            ib = pltpu.async_copy(
                dst_hbm.at[wid, pl.ds((k + 1) * CH, CH)], dst_v.at[pn], isem)

        def inner(jj, carry, k=k, pk=pk):
            j = i32(k * CH) + jj
            p = lax.rem(j, i32(2))
            pnx = lax.rem(j + i32(1), i32(2))

            @pl.when(jj < i32(CH - 1))
            def _():
                pltpu.async_copy(h_hbm.at[src_v.at[pk, jj + i32(1)]],
                                 buf.at[pnx], gsem.at[pnx])

            pltpu.make_async_copy(h_hbm.at[src_v.at[pk, jj]],
                                  buf.at[p], gsem.at[p]).wait()
            pltpu.sync_copy(buf.at[p], acc.at[dst_v.at[pk, jj]], add=True)
            return carry

        lax.fori_loop(i32(0), i32(CH), inner, 0)
        if k + 1 < NCHUNK:
            ia.wait()
            ib.wait()
            j0 = (k + 1) * CH
            pltpu.async_copy(h_hbm.at[src_v.at[pn, i32(0)]],
                             buf.at[i32(j0 % 2)], gsem.at[i32(j0 % 2)])

    plsc.subcore_barrier()

    @pl.when(c == 0)
    def _():
        pltpu.sync_copy(acc.at[rows], out0.at[rows])

    @pl.when(c == 1)
    def _():
        pltpu.sync_copy(acc.at[rows], out1.at[rows])


_agg_call = functools.partial(
    pl.kernel,
    out_type=(
        jax.ShapeDtypeStruct((NP, D), jnp.float32),
        jax.ShapeDtypeStruct((NP, D), jnp.float32),
    ),
    mesh=_sc_mesh(),
    scratch_types=[
        pltpu.VMEM((2, CH, BATCH), jnp.int32),
        pltpu.VMEM((2, CH, BATCH), jnp.int32),
        pltpu.VMEM((2, BATCH, D), jnp.float32),
        pltpu.VMEM_SHARED((NP, D), jnp.float32),
        pltpu.SemaphoreType.DMA((2,)),
        pltpu.SemaphoreType.DMA,
    ],
)(_agg_body)


def _tc1_body(deg0_ref, deg1_ref, x_ref, w_ref, h_ref, dinv_ref):
    d = deg0_ref[0, 0, :] + deg1_ref[0, 0, :] + 1.0
    di = lax.rsqrt(d)
    h = jnp.dot(x_ref[...], w_ref[...], preferred_element_type=jnp.float32,
                precision=lax.Precision.HIGHEST)
    h_ref[...] = di[:, None] * h
    dinv_ref[0, 0, :] = di


def _tc2_body(s0_ref, s1_ref, hp_ref, dinv_ref, b_ref, out_ref):
    di = dinv_ref[0, 0, :][:, None]
    s = s0_ref[...] + s1_ref[...] - hp_ref[...]
    h = jnp.maximum(di * s + b_ref[...][None, :], 0.0)
    out_ref[...] = di * h


def _tc3_body(s0_ref, s1_ref, hp_ref, dinv_ref, w_ref, b_ref, out_ref):
    di = dinv_ref[0, 0, :][:, None]
    a = di * (s0_ref[...] + s1_ref[...] - hp_ref[...])
    out_ref[...] = (
        jnp.dot(a, w_ref[...], preferred_element_type=jnp.float32,
                precision=lax.Precision.HIGHEST)
        + b_ref[...][None, :]
    )


def _row_spec(width):
    return pl.BlockSpec((BLK, width), lambda i: (i, 0))


def _vec_spec():
    return pl.BlockSpec((1, 1, BLK), lambda i: (i, 0, 0))


def _full_spec(r, c):
    return pl.BlockSpec((r, c), lambda i: (0, 0))


_tc1_call = pl.pallas_call(
    _tc1_body,
    grid=(GRID,),
    in_specs=[_vec_spec(), _vec_spec(), _row_spec(D), _full_spec(D, D)],
    out_specs=[_row_spec(D), _vec_spec()],
    out_shape=[
        jax.ShapeDtypeStruct((NP, D), jnp.float32),
        jax.ShapeDtypeStruct((GRID, 1, BLK), jnp.float32),
    ],
)

_tc2_call = pl.pallas_call(
    _tc2_body,
    grid=(GRID,),
    in_specs=[
        _row_spec(D),
        _row_spec(D),
        _row_spec(D),
        _vec_spec(),
        pl.BlockSpec((D,), lambda i: (0,)),
    ],
    out_specs=_row_spec(D),
    out_shape=jax.ShapeDtypeStruct((NP, D), jnp.float32),
)

_tc3_call = pl.pallas_call(
    _tc3_body,
    grid=(GRID,),
    in_specs=[
        _row_spec(D),
        _row_spec(D),
        _row_spec(D),
        _vec_spec(),
        _full_spec(D, D),
        pl.BlockSpec((D,), lambda i: (0,)),
    ],
    out_specs=_row_spec(D),
    out_shape=jax.ShapeDtypeStruct((NP, D), jnp.float32),
)


@jax.jit
def _run(x, src, dst, W1, b1, W_cat, b_cat):
    pad = EP - src.shape[0]
    src_p = jnp.concatenate(
        [src, jnp.zeros((pad,), jnp.int32)]).reshape(NW, NB, BATCH)
    dst_p = jnp.concatenate(
        [dst, N + jnp.arange(pad, dtype=jnp.int32) % (NP - N)],
    ).reshape(NW, NB, BATCH)
    xp = jnp.zeros((NP, D), jnp.float32).at[:N].set(x)

    deg0, deg1 = _deg_call(dst_p)
    hp, dinv = _tc1_call(
        deg0.reshape(GRID, 1, BLK), deg1.reshape(GRID, 1, BLK), xp, W1)
    s1a, s1b = _agg_call(hp, src_p, dst_p)
    hp2 = _tc2_call(s1a, s1b, hp, dinv, b1)
    s2a, s2b = _agg_call(hp2, src_p, dst_p)
    out = _tc3_call(s2a, s2b, hp2, dinv, W_cat, b_cat)
    return out[:N, :64], out[:N, 64:]


def kernel(x, edge_index, W1, b1, W_mu, b_mu, W_var, b_var):
    with jax.enable_x64(False):
        src = edge_index[0].astype(jnp.int32)
        dst = edge_index[1].astype(jnp.int32)
        W_cat = jnp.concatenate([W_mu, W_var], axis=1)
        b_cat = jnp.concatenate([b_mu, b_var], axis=0)
        mu, lv = _run(x.astype(jnp.float32), src, dst,
                      W1.astype(jnp.float32), b1.astype(jnp.float32),
                      W_cat.astype(jnp.float32), b_cat.astype(jnp.float32))
    return mu.astype(jnp.float64), lv.astype(jnp.float64)
